# baseline (speedup 1.0000x reference)
import functools

import jax
import jax.numpy as jnp
from jax import lax
from jax.experimental import pallas as pl
from jax.experimental.pallas import tpu as pltpu
from jax.experimental.pallas import tpu_sc as plsc

N = 10000
E = 320000
H = 128
G = 64
HP = H // 2

SC_TILES = 32
PPT = HP // SC_TILES
CH = 1280
NCH = E // CH
F32 = jnp.float32
I32 = jnp.int32


def _swish(t):
    return t * jax.nn.sigmoid(t)


def _pack_rows(a, b):
    lo = jax.lax.bitcast_convert_type(a.astype(jnp.bfloat16),
                                      jnp.uint16).astype(jnp.uint32)
    hi = jax.lax.bitcast_convert_type(b.astype(jnp.bfloat16),
                                      jnp.uint16).astype(jnp.uint32)
    return jax.lax.bitcast_convert_type(lo | (hi << 16), I32)


def _prologue_kernel(x_ref, wlin_ref, wle_ref, wlo_ref, blin_ref, ble_ref,
                     blo_ref, f1a_ref, f1b_ref, f2a_ref, f2b_ref,
                     xp_ref, xpk_ref, c1_ref, c2_ref):
    xp_ref[...] = _swish(
        jax.lax.dot_general(x_ref[...], wlin_ref[...], (((1,), (1,)), ((), ())),
                            preferred_element_type=F32) + blin_ref[...])
    xta = _swish(
        jax.lax.dot_general(wle_ref[...], x_ref[...], (((1,), (1,)), ((), ())),
                            preferred_element_type=F32) + ble_ref[...])
    xtb = _swish(
        jax.lax.dot_general(wlo_ref[...], x_ref[...], (((1,), (1,)), ((), ())),
                            preferred_element_type=F32) + blo_ref[...])
    xpk_ref[...] = _pack_rows(xta, xtb)
    c1_ref[...] = jax.lax.dot_general(f1b_ref[...], f1a_ref[...],
                                      (((1,), (0,)), ((), ())),
                                      preferred_element_type=F32)
    c2_ref[...] = jax.lax.dot_general(f2b_ref[...], f2a_ref[...],
                                      (((1,), (0,)), ((), ())),
                                      preferred_element_type=F32)


def _edge_factor_kernel(c1e_ref, c1o_ref, c2e_ref, c2o_ref, f1t_in_ref,
                        f2t_in_ref, f1pk_ref, f2pk_ref):
    def dg(c_ref, f_ref):
        return jax.lax.dot_general(c_ref[...], f_ref[...],
                                   (((1,), (0,)), ((), ())),
                                   preferred_element_type=F32)
    f1pk_ref[...] = _pack_rows(dg(c1e_ref, f1t_in_ref), dg(c1o_ref, f1t_in_ref))
    f2pk_ref[...] = _pack_rows(dg(c2e_ref, f2t_in_ref), dg(c2o_ref, f2t_in_ref))


def _sc_agg(xpk, f1pk, f2pk, ei):
    mesh = plsc.VectorSubcoreMesh(core_axis_name="c", subcore_axis_name="s")
    info = plsc.get_sparse_core_info()
    nc = info.num_cores

    @functools.partial(
        pl.kernel, mesh=mesh,
        compiler_params=pltpu.CompilerParams(needs_layout_passes=False),
        out_type=[jax.ShapeDtypeStruct((H * N,), F32),
                  jax.ShapeDtypeStruct((H * N,), F32)],
        scratch_types=[
            pltpu.VMEM((PPT * N,), I32),
            pltpu.VMEM((4 * N,), F32),
            pltpu.VMEM((4 * N,), F32),
            pltpu.VMEM((2, 2, CH), I32),
            pltpu.VMEM((2, PPT, CH), I32),
            pltpu.VMEM((2, PPT, CH), I32),
            pltpu.SemaphoreType.DMA((2,)),
            pltpu.SemaphoreType.DMA((2,)),
            pltpu.SemaphoreType.DMA((2,)),
        ],
    )
    def body(xpk_h, f1pk_h, f2pk_h, ei_h, agg1_h, agg2_h, xsl, acc1, acc2,
             idx, f1b, f2b, sem_i, sem_1, sem_2):
        wid = lax.axis_index("s") * nc + lax.axis_index("c")
        r0 = wid * PPT
        pltpu.sync_copy(xpk_h.at[pl.ds(r0 * N, PPT * N)], xsl)

        @plsc.parallel_loop(0, 4 * N // 16, unroll=8)
        def _zero(i):
            z = jnp.zeros((16,), F32)
            acc1[pl.ds(i * 16, 16)] = z
            acc2[pl.ds(i * 16, 16)] = z

        def start(c, b):
            pltpu.async_copy(ei_h.at[:, pl.ds(c * CH, CH)], idx.at[b],
                             sem_i.at[b])
            pltpu.async_copy(f1pk_h.at[pl.ds(r0, PPT), pl.ds(c * CH, CH)],
                             f1b.at[b], sem_1.at[b])
            pltpu.async_copy(f2pk_h.at[pl.ds(r0, PPT), pl.ds(c * CH, CH)],
                             f2b.at[b], sem_2.at[b])

        def wait(c, b):
            pltpu.make_async_copy(ei_h.at[:, pl.ds(c * CH, CH)],
                                  idx.at[b], sem_i.at[b]).wait()
            pltpu.make_async_copy(f1pk_h.at[pl.ds(r0, PPT), pl.ds(c * CH, CH)],
                                  f1b.at[b], sem_1.at[b]).wait()
            pltpu.make_async_copy(f2pk_h.at[pl.ds(r0, PPT), pl.ds(c * CH, CH)],
                                  f2b.at[b], sem_2.at[b]).wait()

        start(0, 0)

        def chunk_pair(ci, _):
            c0 = ci * 2
            for b in range(2):
                c = c0 + b
                wait(c, b)

                @pl.when(c + 1 < NCH)
                def _():
                    start(c + 1, 1 - b)

                idxb = idx.at[b]
                f1bb = f1b.at[b]
                f2bb = f2b.at[b]

                @plsc.parallel_loop(0, CH // 16, unroll=16)
                def _group(g):
                    s16 = idxb[0, pl.ds(g * 16, 16)]
                    d16 = idxb[1, pl.ds(g * 16, 16)]
                    for r in range(PPT):
                        xw = plsc.load_gather(xsl, [s16 + (r * N)])
                        xlo, xhi = plsc.unpack(
                            plsc.bitcast(xw, jnp.bfloat16),
                            format=plsc.PackFormat.INTERLEAVED)
                        f1w = f1bb[r, pl.ds(g * 16, 16)]
                        f1lo, f1hi = plsc.unpack(
                            plsc.bitcast(f1w, jnp.bfloat16),
                            format=plsc.PackFormat.INTERLEAVED)
                        f2w = f2bb[r, pl.ds(g * 16, 16)]
                        f2lo, f2hi = plsc.unpack(
                            plsc.bitcast(f2w, jnp.bfloat16),
                            format=plsc.PackFormat.INTERLEAVED)
                        dlo = d16 + (2 * r) * N
                        dhi = d16 + (2 * r + 1) * N
                        plsc.addupdate_scatter(acc1, [dlo], xlo * f1lo)
                        plsc.addupdate_scatter(acc1, [dhi], xhi * f1hi)
                        plsc.addupdate_scatter(acc2, [dlo], xlo * f2lo)
                        plsc.addupdate_scatter(acc2, [dhi], xhi * f2hi)
            return 0
        lax.fori_loop(0, NCH // 2, chunk_pair, 0)
        pltpu.sync_copy(acc1, agg1_h.at[pl.ds(r0 * 2 * N, 4 * N)])
        pltpu.sync_copy(acc2, agg2_h.at[pl.ds(r0 * 2 * N, 4 * N)])

    return body(xpk, f1pk, f2pk, ei)


def _epilogue_kernel(xp_ref, a1_ref, a2_ref, batch_ref,
                     wrel1_ref, brel1_ref, wroot1_ref, w1_ref, b1_ref,
                     wrel2_ref, brel2_ref, wroot2_ref, w2_ref, b2_ref,
                     wcat_ref, bcat_ref,
                     lw0_ref, lb0_ref, lw1_ref, lb1_ref, lw2_ref, lb2_ref,
                     normw_ref, normb_ref, normms_ref,
                     wfin_ref, bfin_ref, out_ref):
    def matT(a, w):
        return jax.lax.dot_general(a, w, (((1,), (1,)), ((), ())),
                                   preferred_element_type=F32)

    xp = xp_ref[...]
    h1 = _swish(matT(matT(a1_ref[...], wrel1_ref[...]) + brel1_ref[...]
                     + matT(xp, wroot1_ref[...]), w1_ref[...]) + b1_ref[...])
    h2 = _swish(matT(matT(a2_ref[...], wrel2_ref[...]) + brel2_ref[...]
                     + matT(xp, wroot2_ref[...]), w2_ref[...]) + b2_ref[...])
    wcat = wcat_ref[...]
    h = (matT(h1, wcat[:, :H]) + matT(h2, wcat[:, H:]) + bcat_ref[...] + xp)
    for w_ref, b_ref in ((lw0_ref, lb0_ref), (lw1_ref, lb1_ref),
                         (lw2_ref, lb2_ref)):
        h = _swish(matT(h, w_ref[...]) + b_ref[...]) + h

    onehot = (batch_ref[...] ==
              jax.lax.broadcasted_iota(jnp.int32, (N, G), 1)).astype(F32)
    cnt = jnp.maximum(
        jax.lax.dot_general(jnp.ones((1, N), F32), onehot,
                            (((1,), (0,)), ((), ())),
                            preferred_element_type=F32), 1.0)
    seg = lambda t: jax.lax.dot_general(
        onehot, t, (((0,), (0,)), ((), ())),
        preferred_element_type=F32) / cnt.reshape(G, 1)
    mean = seg(h)
    h = h - normms_ref[...] * jnp.dot(onehot, mean,
                                      preferred_element_type=F32)
    var = seg(h * h)
    std = jnp.sqrt(var + 1e-5)
    h = normw_ref[...] * h / jnp.dot(onehot, std,
                                     preferred_element_type=F32) \
        + normb_ref[...]
    out_ref[...] = matT(h, wfin_ref[...]) + bfin_ref[...]


@jax.jit
def kernel(x, feature1, feature2, edge_index, batch, params):
    p = params
    row = lambda t: t.reshape(1, -1)
    col = lambda t: t.reshape(-1, 1)

    xp, xpk, c1, c2 = pl.pallas_call(
        _prologue_kernel,
        out_shape=[jax.ShapeDtypeStruct((N, H), F32),
                   jax.ShapeDtypeStruct((HP, N), I32),
                   jax.ShapeDtypeStruct((H, 12), F32),
                   jax.ShapeDtypeStruct((H, 6), F32)],
    )(x, p['W_lin'], p['W_lin'][0::2], p['W_lin'][1::2], row(p['b_lin']),
      col(p['b_lin'][0::2]), col(p['b_lin'][1::2]),
      p['W_f1a'], p['W_f1b'], p['W_f2a'], p['W_f2b'])

    EC = 16000
    grid = E // EC
    f1pk, f2pk = pl.pallas_call(
        _edge_factor_kernel,
        grid=(grid,),
        in_specs=[pl.BlockSpec((HP, 12), lambda i: (0, 0)),
                  pl.BlockSpec((HP, 12), lambda i: (0, 0)),
                  pl.BlockSpec((HP, 6), lambda i: (0, 0)),
                  pl.BlockSpec((HP, 6), lambda i: (0, 0)),
                  pl.BlockSpec((12, EC), lambda i: (0, i)),
                  pl.BlockSpec((6, EC), lambda i: (0, i))],
        out_specs=[pl.BlockSpec((HP, EC), lambda i: (0, i)),
                   pl.BlockSpec((HP, EC), lambda i: (0, i))],
        out_shape=[jax.ShapeDtypeStruct((HP, E), I32),
                   jax.ShapeDtypeStruct((HP, E), I32)],
    )(c1[0::2], c1[1::2], c2[0::2], c2[1::2], feature1.T, feature2.T)

    agg1Tf, agg2Tf = _sc_agg(xpk.reshape(-1), f1pk, f2pk, edge_index)
    agg1T = agg1Tf.reshape(H, N)
    agg2T = agg2Tf.reshape(H, N)

    out = pl.pallas_call(
        _epilogue_kernel,
        out_shape=jax.ShapeDtypeStruct((N, H), F32),
    )(xp, agg1T.T, agg2T.T, batch.reshape(N, 1),
      p['W_rel1'], row(p['b_rel1']), p['W_root1'], p['W1'], row(p['b1']),
      p['W_rel2'], row(p['b_rel2']), p['W_root2'], p['W2'], row(p['b2']),
      p['W_cat'], row(p['b_cat']),
      p['lins_W'][0], row(p['lins_b'][0]), p['lins_W'][1],
      row(p['lins_b'][1]), p['lins_W'][2], row(p['lins_b'][2]),
      row(p['norm_w']), row(p['norm_b']), row(p['norm_ms']),
      p['W_final'], row(p['b_final']))
    return out


# speedup vs baseline: 1.1093x; 1.1093x over previous
import functools

import jax
import jax.numpy as jnp
from jax import lax
from jax.experimental import pallas as pl
from jax.experimental.pallas import tpu as pltpu
from jax.experimental.pallas import tpu_sc as plsc

N = 10000
E = 320000
H = 128
G = 64
HP = H // 2

SC_TILES = 32
PPT = HP // SC_TILES
CH = 1280
NCH = E // CH
F32 = jnp.float32
I32 = jnp.int32


def _swish(t):
    return t * jax.nn.sigmoid(t)


def _pack_rows(a, b):
    lo = jax.lax.bitcast_convert_type(a.astype(jnp.bfloat16),
                                      jnp.uint16).astype(jnp.uint32)
    hi = jax.lax.bitcast_convert_type(b.astype(jnp.bfloat16),
                                      jnp.uint16).astype(jnp.uint32)
    return jax.lax.bitcast_convert_type(lo | (hi << 16), I32)


def _prologue_kernel(x_ref, wlin_ref, wle_ref, wlo_ref, blin_ref, ble_ref,
                     blo_ref, f1a_ref, f1b_ref, f2a_ref, f2b_ref,
                     xp_ref, xpk_ref, c1_ref, c2_ref):
    xp_ref[...] = _swish(
        jax.lax.dot_general(x_ref[...], wlin_ref[...], (((1,), (1,)), ((), ())),
                            preferred_element_type=F32) + blin_ref[...])
    xta = _swish(
        jax.lax.dot_general(wle_ref[...], x_ref[...], (((1,), (1,)), ((), ())),
                            preferred_element_type=F32) + ble_ref[...])
    xtb = _swish(
        jax.lax.dot_general(wlo_ref[...], x_ref[...], (((1,), (1,)), ((), ())),
                            preferred_element_type=F32) + blo_ref[...])
    xpk_ref[...] = _pack_rows(xta, xtb)
    c1_ref[...] = jax.lax.dot_general(f1b_ref[...], f1a_ref[...],
                                      (((1,), (0,)), ((), ())),
                                      preferred_element_type=F32)
    c2_ref[...] = jax.lax.dot_general(f2b_ref[...], f2a_ref[...],
                                      (((1,), (0,)), ((), ())),
                                      preferred_element_type=F32)


def _edge_factor_kernel(c1e_ref, c1o_ref, c2e_ref, c2o_ref, f1t_in_ref,
                        f2t_in_ref, f1pk_ref, f2pk_ref):
    def dg(c_ref, f_ref):
        return jax.lax.dot_general(c_ref[...], f_ref[...],
                                   (((1,), (0,)), ((), ())),
                                   preferred_element_type=F32)
    f1pk_ref[...] = _pack_rows(dg(c1e_ref, f1t_in_ref), dg(c1o_ref, f1t_in_ref))
    f2pk_ref[...] = _pack_rows(dg(c2e_ref, f2t_in_ref), dg(c2o_ref, f2t_in_ref))


def _sc_agg(xpk, f1pk, f2pk, ei):
    mesh = plsc.VectorSubcoreMesh(core_axis_name="c", subcore_axis_name="s")
    info = plsc.get_sparse_core_info()
    nc = info.num_cores

    @functools.partial(
        pl.kernel, mesh=mesh,
        compiler_params=pltpu.CompilerParams(needs_layout_passes=False),
        out_type=[jax.ShapeDtypeStruct((H * N,), F32),
                  jax.ShapeDtypeStruct((H * N,), F32)],
        scratch_types=[
            pltpu.VMEM((PPT * N,), I32),
            pltpu.VMEM((4 * N,), F32),
            pltpu.VMEM((4 * N,), F32),
            pltpu.VMEM((2, 2, CH), I32),
            pltpu.VMEM((2, PPT, CH), I32),
            pltpu.VMEM((2, PPT, CH), I32),
            pltpu.SemaphoreType.DMA((2,)),
            pltpu.SemaphoreType.DMA((2,)),
            pltpu.SemaphoreType.DMA((2,)),
        ],
    )
    def body(xpk_h, f1pk_h, f2pk_h, ei_h, agg1_h, agg2_h, xsl, acc1, acc2,
             idx, f1b, f2b, sem_i, sem_1, sem_2):
        wid = lax.axis_index("s") * nc + lax.axis_index("c")
        r0 = wid * PPT
        pltpu.sync_copy(xpk_h.at[pl.ds(r0 * N, PPT * N)], xsl)

        @plsc.parallel_loop(0, 4 * N // 16, unroll=8)
        def _zero(i):
            z = jnp.zeros((16,), F32)
            acc1[pl.ds(i * 16, 16)] = z
            acc2[pl.ds(i * 16, 16)] = z

        def start(c, b):
            pltpu.async_copy(ei_h.at[:, pl.ds(c * CH, CH)], idx.at[b],
                             sem_i.at[b])
            pltpu.async_copy(f1pk_h.at[pl.ds(r0, PPT), pl.ds(c * CH, CH)],
                             f1b.at[b], sem_1.at[b])
            pltpu.async_copy(f2pk_h.at[pl.ds(r0, PPT), pl.ds(c * CH, CH)],
                             f2b.at[b], sem_2.at[b])

        def wait(c, b):
            pltpu.make_async_copy(ei_h.at[:, pl.ds(c * CH, CH)],
                                  idx.at[b], sem_i.at[b]).wait()
            pltpu.make_async_copy(f1pk_h.at[pl.ds(r0, PPT), pl.ds(c * CH, CH)],
                                  f1b.at[b], sem_1.at[b]).wait()
            pltpu.make_async_copy(f2pk_h.at[pl.ds(r0, PPT), pl.ds(c * CH, CH)],
                                  f2b.at[b], sem_2.at[b]).wait()

        start(0, 0)

        def chunk_pair(ci, _):
            c0 = ci * 2
            for b in range(2):
                c = c0 + b
                wait(c, b)

                @pl.when(c + 1 < NCH)
                def _():
                    start(c + 1, 1 - b)

                idxb = idx.at[b]
                f1bb = f1b.at[b]
                f2bb = f2b.at[b]

                @plsc.parallel_loop(0, CH // 16, unroll=8)
                def _group(g):
                    s16 = idxb[0, pl.ds(g * 16, 16)]
                    d16 = idxb[1, pl.ds(g * 16, 16)]
                    for r in range(PPT):
                        xw = plsc.load_gather(xsl, [s16 + (r * N)])
                        xlo, xhi = plsc.unpack(
                            plsc.bitcast(xw, jnp.bfloat16),
                            format=plsc.PackFormat.INTERLEAVED)
                        f1w = f1bb[r, pl.ds(g * 16, 16)]
                        f1lo, f1hi = plsc.unpack(
                            plsc.bitcast(f1w, jnp.bfloat16),
                            format=plsc.PackFormat.INTERLEAVED)
                        f2w = f2bb[r, pl.ds(g * 16, 16)]
                        f2lo, f2hi = plsc.unpack(
                            plsc.bitcast(f2w, jnp.bfloat16),
                            format=plsc.PackFormat.INTERLEAVED)
                        dlo = d16 + (2 * r) * N
                        dhi = d16 + (2 * r + 1) * N
                        plsc.addupdate_scatter(acc1, [dlo], xlo * f1lo)
                        plsc.addupdate_scatter(acc1, [dhi], xhi * f1hi)
                        plsc.addupdate_scatter(acc2, [dlo], xlo * f2lo)
                        plsc.addupdate_scatter(acc2, [dhi], xhi * f2hi)
            return 0
        lax.fori_loop(0, NCH // 2, chunk_pair, 0)
        pltpu.sync_copy(acc1, agg1_h.at[pl.ds(r0 * 2 * N, 4 * N)])
        pltpu.sync_copy(acc2, agg2_h.at[pl.ds(r0 * 2 * N, 4 * N)])

    return body(xpk, f1pk, f2pk, ei)


def _epilogue_kernel(xp_ref, a1_ref, a2_ref, batch_ref,
                     wrel1_ref, brel1_ref, wroot1_ref, w1_ref, b1_ref,
                     wrel2_ref, brel2_ref, wroot2_ref, w2_ref, b2_ref,
                     wcat_ref, bcat_ref,
                     lw0_ref, lb0_ref, lw1_ref, lb1_ref, lw2_ref, lb2_ref,
                     normw_ref, normb_ref, normms_ref,
                     wfin_ref, bfin_ref, out_ref):
    def matT(a, w):
        return jax.lax.dot_general(a, w, (((1,), (1,)), ((), ())),
                                   preferred_element_type=F32)

    def matT0(aT, w):
        return jax.lax.dot_general(aT, w, (((0,), (1,)), ((), ())),
                                   preferred_element_type=F32)

    xp = xp_ref[...]
    h1 = _swish(matT(matT0(a1_ref[...], wrel1_ref[...]) + brel1_ref[...]
                     + matT(xp, wroot1_ref[...]), w1_ref[...]) + b1_ref[...])
    h2 = _swish(matT(matT0(a2_ref[...], wrel2_ref[...]) + brel2_ref[...]
                     + matT(xp, wroot2_ref[...]), w2_ref[...]) + b2_ref[...])
    wcat = wcat_ref[...]
    h = (matT(h1, wcat[:, :H]) + matT(h2, wcat[:, H:]) + bcat_ref[...] + xp)
    for w_ref, b_ref in ((lw0_ref, lb0_ref), (lw1_ref, lb1_ref),
                         (lw2_ref, lb2_ref)):
        h = _swish(matT(h, w_ref[...]) + b_ref[...]) + h

    onehot = (batch_ref[...] ==
              jax.lax.broadcasted_iota(jnp.int32, (N, G), 1)).astype(F32)
    cnt = jnp.maximum(
        jax.lax.dot_general(jnp.ones((1, N), F32), onehot,
                            (((1,), (0,)), ((), ())),
                            preferred_element_type=F32), 1.0)
    seg = lambda t: jax.lax.dot_general(
        onehot, t, (((0,), (0,)), ((), ())),
        preferred_element_type=F32) / cnt.reshape(G, 1)
    mean = seg(h)
    h = h - normms_ref[...] * jnp.dot(onehot, mean,
                                      preferred_element_type=F32)
    var = seg(h * h)
    std = jnp.sqrt(var + 1e-5)
    h = normw_ref[...] * h / jnp.dot(onehot, std,
                                     preferred_element_type=F32) \
        + normb_ref[...]
    out_ref[...] = matT(h, wfin_ref[...]) + bfin_ref[...]


@jax.jit
def kernel(x, feature1, feature2, edge_index, batch, params):
    p = params
    row = lambda t: t.reshape(1, -1)
    col = lambda t: t.reshape(-1, 1)

    xp, xpk, c1, c2 = pl.pallas_call(
        _prologue_kernel,
        out_shape=[jax.ShapeDtypeStruct((N, H), F32),
                   jax.ShapeDtypeStruct((HP, N), I32),
                   jax.ShapeDtypeStruct((H, 12), F32),
                   jax.ShapeDtypeStruct((H, 6), F32)],
    )(x, p['W_lin'], p['W_lin'][0::2], p['W_lin'][1::2], row(p['b_lin']),
      col(p['b_lin'][0::2]), col(p['b_lin'][1::2]),
      p['W_f1a'], p['W_f1b'], p['W_f2a'], p['W_f2b'])

    EC = 16000
    grid = E // EC
    f1pk, f2pk = pl.pallas_call(
        _edge_factor_kernel,
        grid=(grid,),
        in_specs=[pl.BlockSpec((HP, 12), lambda i: (0, 0)),
                  pl.BlockSpec((HP, 12), lambda i: (0, 0)),
                  pl.BlockSpec((HP, 6), lambda i: (0, 0)),
                  pl.BlockSpec((HP, 6), lambda i: (0, 0)),
                  pl.BlockSpec((12, EC), lambda i: (0, i)),
                  pl.BlockSpec((6, EC), lambda i: (0, i))],
        out_specs=[pl.BlockSpec((HP, EC), lambda i: (0, i)),
                   pl.BlockSpec((HP, EC), lambda i: (0, i))],
        out_shape=[jax.ShapeDtypeStruct((HP, E), I32),
                   jax.ShapeDtypeStruct((HP, E), I32)],
    )(c1[0::2], c1[1::2], c2[0::2], c2[1::2], feature1.T, feature2.T)

    agg1Tf, agg2Tf = _sc_agg(xpk.reshape(-1), f1pk, f2pk, edge_index)
    agg1T = agg1Tf.reshape(H, N)
    agg2T = agg2Tf.reshape(H, N)

    out = pl.pallas_call(
        _epilogue_kernel,
        out_shape=jax.ShapeDtypeStruct((N, H), F32),
    )(xp, agg1T, agg2T, batch.reshape(N, 1),
      p['W_rel1'], row(p['b_rel1']), p['W_root1'], p['W1'], row(p['b1']),
      p['W_rel2'], row(p['b_rel2']), p['W_root2'], p['W2'], row(p['b2']),
      p['W_cat'], row(p['b_cat']),
      p['lins_W'][0], row(p['lins_b'][0]), p['lins_W'][1],
      row(p['lins_b'][1]), p['lins_W'][2], row(p['lins_b'][2]),
      row(p['norm_w']), row(p['norm_b']), row(p['norm_ms']),
      p['W_final'], row(p['b_final']))
    return out
